# baseline (speedup 1.0000x reference)
import functools

import jax
import jax.numpy as jnp
from jax import lax
from jax.experimental import pallas as pl
from jax.experimental.pallas import tpu as pltpu
from jax.experimental.pallas import tpu_sc as plsc

LATENT = 256
K = 8192
BETA = 0.25
M = 8 * 56 * 56
RBLK = 256
NBLK = M // RBLK
KWIN = 2048
KEDGES = tuple(range(0, K, KWIN)) + (K,)

NW = 32
BPW = M // NW
SUB = 112
NSUB = BPW // SUB


def _vq_tc_body(flat_ref, cb_ref, idx_ref, ppl_ref, loss_ref,
                cnorm_ref, counts_ref, losssum_ref):
    i = pl.program_id(0)

    @pl.when(i == 0)
    def _init():
        cb = cb_ref[...]
        cnorm_ref[...] = jnp.sum(cb * cb, axis=1)[None, :]
        counts_ref[...] = jnp.zeros_like(counts_ref)
        losssum_ref[...] = jnp.zeros_like(losssum_ref)

    flat = flat_ref[...]
    mm = lax.dot_general(flat, cb_ref[...], (((1,), (1,)), ((), ())),
                         preferred_element_type=jnp.float32)
    znorm = jnp.sum(flat * flat, axis=1, keepdims=True)
    dist = (znorm - 2.0 * mm) + cnorm_ref[...]
    lane = lax.broadcasted_iota(jnp.int32, dist.shape, 1)

    def _rnd(x):
        xi = lax.bitcast_convert_type(x, jnp.int32)
        xi = (xi + 0x7FFF + ((xi >> 16) & 1)) & jnp.int32(-65536)
        return lax.bitcast_convert_type(xi, jnp.float32)

    chunks = []
    inf = jnp.float32(jnp.inf)
    for (s, e) in zip(KEDGES[:-1], KEDGES[1:]):
        dm = jnp.where((lane >= s) & (lane < e), dist, inf)
        m = jnp.min(dm, axis=1, keepdims=True)
        a = jnp.min(jnp.where(dm == m, lane, K), axis=1, keepdims=True)
        chunks.append((m, a))
    (m1, a1) = chunks[0]
    v = _rnd(m1)
    acc_a, acc_m = a1, m1
    for m2_, a2_ in chunks[1:]:
        keep = v <= m2_
        acc_a = jnp.where(keep, acc_a, a2_)
        acc_m = jnp.where(keep, acc_m, m2_)
        v = jnp.where(keep, v, _rnd(m2_))
    idx = acc_a[:, 0]
    idx_ref[...] = idx[None, None, :]
    onehot = (lane == idx[:, None]).astype(jnp.float32)
    counts_ref[...] += jnp.sum(onehot, axis=0, keepdims=True)
    losssum_ref[...] += jnp.sum(acc_m, axis=(0, 1), keepdims=True)

    @pl.when(i == NBLK - 1)
    def _fin():
        p = counts_ref[...] * (1.0 / M)
        ent = jnp.sum(p * jnp.log(p + 1e-10), axis=(0, 1), keepdims=True)
        ppl_ref[...] = jnp.exp(-ent)
        loss_ref[...] = BETA * (losssum_ref[...] * (1.0 / (M * LATENT)))


def _vq_argmin(flat, codebook):
    return pl.pallas_call(
        _vq_tc_body,
        grid=(NBLK,),
        in_specs=[
            pl.BlockSpec((RBLK, LATENT), lambda i: (i, 0)),
            pl.BlockSpec((K, LATENT), lambda i: (0, 0)),
        ],
        out_specs=[
            pl.BlockSpec((1, 1, RBLK), lambda i: (i, 0, 0)),
            pl.BlockSpec((1, 1), lambda i: (0, 0)),
            pl.BlockSpec((1, 1), lambda i: (0, 0)),
        ],
        out_shape=[
            jax.ShapeDtypeStruct((NBLK, 1, RBLK), jnp.int32),
            jax.ShapeDtypeStruct((1, 1), jnp.float32),
            jax.ShapeDtypeStruct((1, 1), jnp.float32),
        ],
        scratch_shapes=[
            pltpu.VMEM((1, K), jnp.float32),
            pltpu.VMEM((1, K), jnp.float32),
            pltpu.VMEM((1, 1), jnp.float32),
        ],
    )(flat, codebook)


@functools.cache
def _make_sc_gather():
    @functools.partial(
        pl.kernel,
        out_type=jax.ShapeDtypeStruct((M, LATENT), jnp.float32),
        mesh=plsc.VectorSubcoreMesh(core_axis_name="c", subcore_axis_name="s"),
        scratch_types=[
            pltpu.VMEM((SUB,), jnp.int32),
            pltpu.VMEM((SUB, LATENT), jnp.float32),
            pltpu.SemaphoreType.DMA,
        ],
    )
    def _sc_gather(table_hbm, idx_hbm, out_hbm, idx_v, rows_v, sem):
        wid = lax.axis_index("s") * 2 + lax.axis_index("c")
        base = wid * BPW
        for j in range(NSUB):
            off = base + j * SUB
            pltpu.sync_copy(idx_hbm.at[pl.ds(off, SUB)], idx_v)
            pltpu.async_copy(table_hbm.at[idx_v], rows_v, sem).wait()
            pltpu.sync_copy(rows_v, out_hbm.at[pl.ds(off, SUB)])

    return _sc_gather


def _conv2(x, W, b, stride):
    y = lax.conv_general_dilated(x, W, (stride, stride), 'SAME',
                                 dimension_numbers=('NHWC', 'HWIO', 'NHWC'))
    return y + b


def _deconv2(x, W, b, stride):
    y = lax.conv_transpose(x, W, (stride, stride), 'SAME',
                           dimension_numbers=('NHWC', 'HWIO', 'NHWC'))
    return y + b


def kernel(x, W1, b1, W2, b2, W3, b3, codebook, D1, bd1, D2, bd2, D3, bd3,
           training):
    ze = jax.nn.relu(_conv2(x, W1, b1, 2))
    ze = jax.nn.relu(_conv2(ze, W2, b2, 2))
    ze = _conv2(ze, W3, b3, 1)

    flat = ze.reshape(-1, LATENT)
    idx, ppl, loss = _vq_argmin(flat, codebook)
    idx = idx.reshape(M)
    zq = _make_sc_gather()(codebook, idx).reshape(ze.shape)

    zq_st = ze + lax.stop_gradient(zq - ze)

    h = jax.nn.relu(_conv2(zq_st, D1, bd1, 1))
    h = jax.nn.relu(_deconv2(h, D2, bd2, 2))
    recon = _deconv2(h, D3, bd3, 2)
    return (recon, ppl.reshape(()), loss.reshape(()))


# speedup vs baseline: 1.1375x; 1.1375x over previous
import functools

import jax
import jax.numpy as jnp
from jax import lax
from jax.experimental import pallas as pl
from jax.experimental.pallas import tpu as pltpu
from jax.experimental.pallas import tpu_sc as plsc

LATENT = 256
K = 8192
BETA = 0.25
M = 8 * 56 * 56
RBLK = 256
NBLK = M // RBLK
KWIN = 2048
KEDGES = tuple(range(0, K, KWIN)) + (K,)

NW = 32
BPW = M // NW
SUB = 112
NSUB = BPW // SUB


def _vq_tc_body(flat_ref, cb_ref, idx_ref, ppl_ref, loss_ref,
                cnorm_ref, counts_ref, losssum_ref):
    i = pl.program_id(0)

    @pl.when(i == 0)
    def _init():
        cb = cb_ref[...]
        cnorm_ref[...] = jnp.sum(cb * cb, axis=1)[None, :]
        counts_ref[...] = jnp.zeros_like(counts_ref)
        losssum_ref[...] = jnp.zeros_like(losssum_ref)

    flat = flat_ref[...]
    mm = lax.dot_general(flat, cb_ref[...], (((1,), (1,)), ((), ())),
                         preferred_element_type=jnp.float32)
    znorm = jnp.sum(flat * flat, axis=1, keepdims=True)
    dist = (znorm - 2.0 * mm) + cnorm_ref[...]
    lane = lax.broadcasted_iota(jnp.int32, dist.shape, 1)

    def _rnd(x):
        xi = lax.bitcast_convert_type(x, jnp.int32)
        xi = (xi + 0x7FFF + ((xi >> 16) & 1)) & jnp.int32(-65536)
        return lax.bitcast_convert_type(xi, jnp.float32)

    ms = [jnp.min(dist[:, s:e], axis=1, keepdims=True)
          for s, e in zip(KEDGES[:-1], KEDGES[1:])]
    v = _rnd(ms[0])
    acc_m = ms[0]
    acc_w = jnp.zeros_like(ms[0], dtype=jnp.int32)
    for w, mw in enumerate(ms[1:], start=1):
        keep = v <= mw
        acc_w = jnp.where(keep, acc_w, w)
        acc_m = jnp.where(keep, acc_m, mw)
        v = jnp.where(keep, v, _rnd(mw))
    hit = (dist == acc_m) & ((lane >> 11) == acc_w)
    idx = jnp.min(jnp.where(hit, lane, K), axis=1)
    idx_ref[...] = idx[None, None, :]
    onehot = (lane == idx[:, None]).astype(jnp.float32)
    counts_ref[...] += jnp.sum(onehot, axis=0, keepdims=True)
    losssum_ref[...] += jnp.sum(acc_m, axis=(0, 1), keepdims=True)

    @pl.when(i == NBLK - 1)
    def _fin():
        p = counts_ref[...] * (1.0 / M)
        ent = jnp.sum(p * jnp.log(p + 1e-10), axis=(0, 1), keepdims=True)
        ppl_ref[...] = jnp.exp(-ent)
        loss_ref[...] = BETA * (losssum_ref[...] * (1.0 / (M * LATENT)))


def _vq_argmin(flat, codebook):
    return pl.pallas_call(
        _vq_tc_body,
        grid=(NBLK,),
        in_specs=[
            pl.BlockSpec((RBLK, LATENT), lambda i: (i, 0)),
            pl.BlockSpec((K, LATENT), lambda i: (0, 0)),
        ],
        out_specs=[
            pl.BlockSpec((1, 1, RBLK), lambda i: (i, 0, 0)),
            pl.BlockSpec((1, 1), lambda i: (0, 0)),
            pl.BlockSpec((1, 1), lambda i: (0, 0)),
        ],
        out_shape=[
            jax.ShapeDtypeStruct((NBLK, 1, RBLK), jnp.int32),
            jax.ShapeDtypeStruct((1, 1), jnp.float32),
            jax.ShapeDtypeStruct((1, 1), jnp.float32),
        ],
        scratch_shapes=[
            pltpu.VMEM((1, K), jnp.float32),
            pltpu.VMEM((1, K), jnp.float32),
            pltpu.VMEM((1, 1), jnp.float32),
        ],
    )(flat, codebook)


@functools.cache
def _make_sc_gather():
    @functools.partial(
        pl.kernel,
        out_type=jax.ShapeDtypeStruct((M, LATENT), jnp.float32),
        mesh=plsc.VectorSubcoreMesh(core_axis_name="c", subcore_axis_name="s"),
        scratch_types=[
            pltpu.VMEM((SUB,), jnp.int32),
            pltpu.VMEM((SUB, LATENT), jnp.float32),
            pltpu.SemaphoreType.DMA,
        ],
    )
    def _sc_gather(table_hbm, idx_hbm, out_hbm, idx_v, rows_v, sem):
        wid = lax.axis_index("s") * 2 + lax.axis_index("c")
        base = wid * BPW
        for j in range(NSUB):
            off = base + j * SUB
            pltpu.sync_copy(idx_hbm.at[pl.ds(off, SUB)], idx_v)
            pltpu.async_copy(table_hbm.at[idx_v], rows_v, sem).wait()
            pltpu.sync_copy(rows_v, out_hbm.at[pl.ds(off, SUB)])

    return _sc_gather


def _conv2(x, W, b, stride):
    y = lax.conv_general_dilated(x, W, (stride, stride), 'SAME',
                                 dimension_numbers=('NHWC', 'HWIO', 'NHWC'))
    return y + b


def _deconv2(x, W, b, stride):
    y = lax.conv_transpose(x, W, (stride, stride), 'SAME',
                           dimension_numbers=('NHWC', 'HWIO', 'NHWC'))
    return y + b


def kernel(x, W1, b1, W2, b2, W3, b3, codebook, D1, bd1, D2, bd2, D3, bd3,
           training):
    ze = jax.nn.relu(_conv2(x, W1, b1, 2))
    ze = jax.nn.relu(_conv2(ze, W2, b2, 2))
    ze = _conv2(ze, W3, b3, 1)

    flat = ze.reshape(-1, LATENT)
    idx, ppl, loss = _vq_argmin(flat, codebook)
    idx = idx.reshape(M)
    zq = _make_sc_gather()(codebook, idx).reshape(ze.shape)

    zq_st = ze + lax.stop_gradient(zq - ze)

    h = jax.nn.relu(_conv2(zq_st, D1, bd1, 1))
    h = jax.nn.relu(_deconv2(h, D2, bd2, 2))
    recon = _deconv2(h, D3, bd3, 2)
    return (recon, ppl.reshape(()), loss.reshape(()))
